# baseline (speedup 1.0000x reference)
import functools

import jax
import jax.numpy as jnp
from jax import lax
from jax.experimental import pallas as pl
from jax.experimental.pallas import tpu as pltpu
from jax.experimental.pallas import tpu_sc as plsc

NC = 2
NS = 16
NW = NC * NS
LANES = 16
DP = 128
BW = 128
NBUF = 3
NVEC = BW // LANES


def _row_has_zero(idx2d, r):
    mn = idx2d[r, pl.ds(0, LANES)]
    for i in range(1, NVEC):
        mn = jnp.minimum(mn, idx2d[r, pl.ds(i * LANES, LANES)])
    zm = jnp.where(mn == 0, 1, 0)
    flag = zm[0]
    for j in range(1, LANES):
        flag = flag | zm[j]
    return flag != 0


def _zero_pad_rows(idx2d, r, rowbuf):
    d = rowbuf.shape[-1]
    zeros = jnp.zeros((LANES,), jnp.float32)

    def fix_group(i, _):
        v = idx2d[r, pl.ds(i * LANES, LANES)]
        zm = jnp.where(v == 0, 1, 0)
        for j in range(LANES):

            @pl.when(zm[j] != 0)
            def _():
                row = i * LANES + j
                for q in range(d // LANES):
                    rowbuf[row, pl.ds(q * LANES, LANES)] = zeros

        return 0

    lax.fori_loop(0, NVEC, fix_group, 0)


def _transpose_block(rowbuf, tbuf, d):
    viota = [lax.iota(jnp.int32, LANES) + v * LANES for v in range(NVEC)]
    for c in range(d):
        cols = jnp.full((LANES,), c, jnp.int32)
        for v in range(NVEC):
            tbuf[c // 8, c % 8, pl.ds(v * LANES, LANES)] = plsc.load_gather(
                rowbuf, [viota[v], cols])


def _emb_body(n_t, d, item_t, user_t, ids_t, out,
              idxv, rows, tbufs, gsem, ssem):
    wid = lax.axis_index("s") * NC + lax.axis_index("c")

    pltpu.sync_copy(ids_t.at[:, pl.ds(wid * BW, BW)], idxv)

    def gather(t, b):
        return pltpu.make_async_copy(
            item_t.at[idxv.at[t]], rows[b], gsem[b])

    def scatter(t, b):
        return pltpu.make_async_copy(
            tbufs[b], out.at[t, :, wid], ssem[b])

    pltpu.make_async_copy(user_t.at[idxv.at[0]], rows[0], gsem[0]).start()
    for b in range(1, NBUF):
        gather(b, b).start()

    def group(g, _):
        for b in range(NBUF):
            t = g * NBUF + b
            gather(t, b).wait()

            @pl.when(_row_has_zero(idxv, t))
            def _():
                _zero_pad_rows(idxv, t, rows[b])

            @pl.when(t >= NBUF)
            def _():
                scatter(t - NBUF, b).wait()

            _transpose_block(rows[b], tbufs[b], d)
            scatter(t, b).start()

            @pl.when(t + NBUF < n_t)
            def _():
                gather(t + NBUF, b).start()

        return 0

    lax.fori_loop(0, n_t // NBUF, group, 0)
    for b in range(NBUF):
        scatter(n_t - NBUF + b, b).wait()


def kernel(item_ids, item_actions, user_id, feat, item_table, user_table):
    B, L = item_ids.shape
    D = item_table.shape[1]
    n_t = L + 1

    ni = item_table.shape[0]
    nu = user_table.shape[0]
    ni_p = (ni + 7) // 8 * 8
    nu_p = (nu + 7) // 8 * 8
    item_p = jnp.pad(item_table, ((0, ni_p - ni), (0, DP - D)))
    user_p = jnp.pad(user_table, ((0, nu_p - nu), (0, DP - D)))

    nt_p = (n_t + 7) // 8 * 8
    ids_t = jnp.concatenate(
        [user_id[None, :], item_ids.T,
         jnp.zeros((nt_p - n_t, B), jnp.int32)], axis=0)

    body = functools.partial(_emb_body, n_t, D)
    grid_kernel = pl.kernel(
        body,
        out_type=jax.ShapeDtypeStruct((n_t, D // 8, B // BW, 8, BW),
                                      jnp.float32),
        mesh=plsc.VectorSubcoreMesh(core_axis_name="c", subcore_axis_name="s"),
        compiler_params=pltpu.CompilerParams(use_tc_tiling_on_sc=True,
                                             needs_layout_passes=False),
        scratch_types=dict(
            idxv=pltpu.VMEM((nt_p, BW), jnp.int32),
            rows=[pltpu.VMEM((BW, DP), jnp.float32) for _ in range(NBUF)],
            tbufs=[pltpu.VMEM((D // 8, 8, BW), jnp.float32)
                   for _ in range(NBUF)],
            gsem=[pltpu.SemaphoreType.DMA for _ in range(NBUF)],
            ssem=[pltpu.SemaphoreType.DMA for _ in range(NBUF)],
        ),
    )
    out5d = grid_kernel(item_p, user_p, ids_t)
    return out5d.transpose(2, 4, 0, 1, 3).reshape(B, n_t, D)


# speedup vs baseline: 1.9634x; 1.9634x over previous
import functools

import jax
import jax.numpy as jnp
from jax import lax
from jax.experimental import pallas as pl
from jax.experimental.pallas import tpu as pltpu
from jax.experimental.pallas import tpu_sc as plsc

NC = 2
NS = 16
NW = NC * NS
LANES = 16
DP = 128
IW = 128
NBUF = 4
NVEC = IW // LANES


def _row_has_zero(idx2d, r):
    mn = idx2d[r, pl.ds(0, LANES)]
    for i in range(1, NVEC):
        mn = jnp.minimum(mn, idx2d[r, pl.ds(i * LANES, LANES)])
    zm = jnp.where(mn == 0, 1, 0)
    flag = zm[0]
    for j in range(1, LANES):
        flag = flag | zm[j]
    return flag != 0


def _zero_pad_rows(idx2d, r, rowbuf):
    d = rowbuf.shape[-1]
    zeros = jnp.zeros((LANES,), jnp.float32)

    def fix_group(i, _):
        v = idx2d[r, pl.ds(i * LANES, LANES)]
        zm = jnp.where(v == 0, 1, 0)
        for j in range(LANES):

            @pl.when(zm[j] != 0)
            def _():
                row = i * LANES + j
                for q in range(d // LANES):
                    rowbuf[row, pl.ds(q * LANES, LANES)] = zeros

        return 0

    lax.fori_loop(0, NVEC, fix_group, 0)


def _emb_body(n_rows, item_t, user_t, src_i, dst_i, src_u, dst_u, out,
              srcv, dstv, srcuv, dstuv, rows,
              gsem, ssem, usem):
    wid = lax.axis_index("s") * NC + lax.axis_index("c")

    pltpu.sync_copy(src_i.at[wid], srcv)
    pltpu.sync_copy(dst_i.at[wid], dstv)
    pltpu.sync_copy(src_u.at[wid], srcuv)
    pltpu.sync_copy(dst_u.at[wid], dstuv)

    pltpu.async_copy(user_t.at[srcuv.at[0]], rows[0], usem).wait()

    @pl.when(_row_has_zero(srcuv, 0))
    def _():
        _zero_pad_rows(srcuv, 0, rows[0])

    pltpu.async_copy(rows[0], out.at[dstuv], usem).wait()

    def group(g, _):
        gathers = []
        for b in range(NBUF):
            c = g * NBUF + b

            @pl.when(g > 0)
            def _():
                pltpu.make_async_copy(
                    rows[b], out.at[dstv.at[c - NBUF]], ssem[b]).wait()

            h = pltpu.make_async_copy(
                item_t.at[srcv.at[c]], rows[b], gsem[b])
            h.start()
            gathers.append(h)
        for b in range(NBUF):
            c = g * NBUF + b
            gathers[b].wait()

            @pl.when(_row_has_zero(srcv, c))
            def _():
                _zero_pad_rows(srcv, c, rows[b])

            pltpu.make_async_copy(
                rows[b], out.at[dstv.at[c]], ssem[b]).start()
        return 0

    lax.fori_loop(0, n_rows // NBUF, group, 0)
    for b in range(NBUF):
        pltpu.make_async_copy(
            rows[b], out.at[dstv.at[n_rows - NBUF + b]], ssem[b]).wait()


def kernel(item_ids, item_actions, user_id, feat, item_table, user_table):
    B, L = item_ids.shape
    D = item_table.shape[1]
    bpw = B // NW
    n_rows = (bpw * L) // IW

    ni = item_table.shape[0]
    nu = user_table.shape[0]
    ni_p = (ni + 7) // 8 * 8
    nu_p = (nu + 7) // 8 * 8
    item_p = jnp.pad(item_table, ((0, ni_p - ni), (0, DP - D)))
    user_p = jnp.pad(user_table, ((0, nu_p - nu), (0, DP - D)))

    src_i = item_ids.reshape(NW, n_rows, IW)
    tp = (L + 1 + 7) // 8 * 8
    dst_i = (jnp.arange(B, dtype=jnp.int32)[:, None] * tp + 1
             + jnp.arange(L, dtype=jnp.int32)[None, :]).reshape(NW, n_rows, IW)
    src_u = user_id.reshape(NW, 1, bpw)
    dst_u = (jnp.arange(B, dtype=jnp.int32) * tp).reshape(NW, bpw)

    body = functools.partial(_emb_body, n_rows)
    grid_kernel = pl.kernel(
        body,
        out_type=jax.ShapeDtypeStruct((B * tp, DP), jnp.float32),
        mesh=plsc.VectorSubcoreMesh(core_axis_name="c", subcore_axis_name="s"),
        compiler_params=pltpu.CompilerParams(use_tc_tiling_on_sc=True),
        scratch_types=dict(
            srcv=pltpu.VMEM((n_rows, IW), jnp.int32),
            dstv=pltpu.VMEM((n_rows, IW), jnp.int32),
            srcuv=pltpu.VMEM((1, bpw), jnp.int32),
            dstuv=pltpu.VMEM((bpw,), jnp.int32),
            rows=[pltpu.VMEM((IW, DP), jnp.float32) for _ in range(NBUF)],
            gsem=[pltpu.SemaphoreType.DMA for _ in range(NBUF)],
            ssem=[pltpu.SemaphoreType.DMA for _ in range(NBUF)],
            usem=pltpu.SemaphoreType.DMA,
        ),
    )
    out = grid_kernel(item_p, user_p, src_i, dst_i, src_u, dst_u)
    return out.reshape(B, tp, DP)[:, :L + 1, :D]
